# baseline (speedup 1.0000x reference)
import jax
import jax.numpy as jnp
from jax import lax
from jax.experimental import pallas as pl
from jax.experimental.pallas import tpu as pltpu
from jax.experimental.pallas import tpu_sc as plsc

_VOCAB = 128
_HID = 64
_LANES = 128
_ROWS_PER_BLK = 8
_NC = 2
_NS = 16
_NW = _NC * _NS


def _gate_table_body(table_ref, w_ref, b_ref, sig_ref):
    t = table_ref[...]
    w = w_ref[...]
    logits = jnp.sum(t * w, axis=1) + b_ref[0, 0]
    sig_ref[...] = jax.nn.sigmoid(logits)[None, :]


def _sc_body(seq_hbm, table_hbm, sig_hbm, h_hbm, soft_hbm,
             idx_v, rows_v, soft_v, sem):
    wid = lax.axis_index("s") * _NC + lax.axis_index("c")
    n_rows = seq_hbm.shape[0]
    rows_per_w = n_rows // _NW
    n_blk = rows_per_w // _ROWS_PER_BLK

    def blk_body(b, carry):
        row0 = wid * rows_per_w + b * _ROWS_PER_BLK
        pltpu.sync_copy(seq_hbm.at[pl.ds(row0, _ROWS_PER_BLK)], idx_v)
        descs = []
        for k in range(_ROWS_PER_BLK):
            descs.append(pltpu.async_copy(
                table_hbm.at[idx_v.at[k]],
                rows_v.at[pl.ds(k * _LANES, _LANES)], sem))
            descs.append(pltpu.async_copy(
                sig_hbm.at[idx_v.at[k]], soft_v.at[k], sem))
        for d in descs:
            d.wait()
        pltpu.sync_copy(
            rows_v, h_hbm.at[pl.ds(row0 * _LANES, _ROWS_PER_BLK * _LANES)])
        pltpu.sync_copy(soft_v, soft_hbm.at[pl.ds(row0, _ROWS_PER_BLK)])
        return carry

    lax.fori_loop(0, n_blk, blk_body, 0)


def kernel(seq, embed_table, gate_w, gate_b):
    B, L = seq.shape
    n = B * L
    seq2d = seq.reshape(n // _LANES, _LANES).astype(jnp.int32)

    sig = pl.pallas_call(
        _gate_table_body,
        out_shape=jax.ShapeDtypeStruct((1, _VOCAB), jnp.float32),
    )(embed_table, gate_w, gate_b.reshape(1, 1))
    sig1d = sig.reshape(_VOCAB)

    mesh = plsc.VectorSubcoreMesh(core_axis_name="c", subcore_axis_name="s",
                                  num_cores=_NC, num_subcores=_NS)
    h_flat, soft2d = pl.kernel(
        _sc_body,
        out_type=[
            jax.ShapeDtypeStruct((n, _HID), jnp.float32),
            jax.ShapeDtypeStruct((n // _LANES, _LANES), jnp.float32),
        ],
        mesh=mesh,
        scratch_types=[
            pltpu.VMEM((_ROWS_PER_BLK, _LANES), jnp.int32),
            pltpu.VMEM((_ROWS_PER_BLK * _LANES, _HID), jnp.float32),
            pltpu.VMEM((_ROWS_PER_BLK, _LANES), jnp.float32),
            pltpu.SemaphoreType.DMA,
        ],
        compiler_params=pltpu.CompilerParams(use_tc_tiling_on_sc=False),
    )(seq2d, embed_table, sig1d)

    h = h_flat.reshape(B, L, _HID)
    soft = soft2d.reshape(B, L)
    return (soft, h)


# speedup vs baseline: 3.3281x; 3.3281x over previous
import jax
import jax.numpy as jnp
from jax import lax
from jax.experimental import pallas as pl
from jax.experimental.pallas import tpu as pltpu
from jax.experimental.pallas import tpu_sc as plsc

_VOCAB = 128
_HID = 64
_BLK = 1024
_NC = 2
_NS = 16
_NW = _NC * _NS


def _gate_table_body(table_ref, w_ref, b_ref, sig_ref):
    t = table_ref[...]
    w = w_ref[...]
    logits = jnp.sum(t * w, axis=1) + b_ref[0, 0]
    sig_ref[...] = jax.nn.sigmoid(logits)[None, :]


def _sc_body(seq_hbm, table_hbm, sig_hbm, h_hbm, soft_hbm,
             idx_v, rows_v, soft_v, sig_v, sem):
    wid = lax.axis_index("s") * _NC + lax.axis_index("c")
    n_idx = seq_hbm.shape[0]
    per_w = n_idx // _NW
    n_blk = per_w // _BLK

    pltpu.sync_copy(sig_hbm, sig_v)

    def blk_body(b, carry):
        base = wid * per_w + b * _BLK
        pltpu.sync_copy(seq_hbm.at[pl.ds(base, _BLK)], idx_v)
        g = pltpu.async_copy(table_hbm.at[idx_v], rows_v, sem)
        for t in range(_BLK // 16):
            iv = idx_v[pl.ds(t * 16, 16)]
            soft_v[pl.ds(t * 16, 16)] = plsc.load_gather(sig_v, [iv])
        g.wait()
        pltpu.sync_copy(rows_v, h_hbm.at[pl.ds(base, _BLK)])
        pltpu.sync_copy(soft_v, soft_hbm.at[pl.ds(base, _BLK)])
        return carry

    lax.fori_loop(0, n_blk, blk_body, 0)


def kernel(seq, embed_table, gate_w, gate_b):
    B, L = seq.shape
    n = B * L
    seq1d = seq.reshape(n).astype(jnp.int32)

    sig = pl.pallas_call(
        _gate_table_body,
        out_shape=jax.ShapeDtypeStruct((1, _VOCAB), jnp.float32),
    )(embed_table, gate_w, gate_b.reshape(1, 1))
    sig1d = sig.reshape(_VOCAB)

    mesh = plsc.VectorSubcoreMesh(core_axis_name="c", subcore_axis_name="s",
                                  num_cores=_NC, num_subcores=_NS)
    h_flat, soft1d = pl.kernel(
        _sc_body,
        out_type=[
            jax.ShapeDtypeStruct((n, _HID), jnp.float32),
            jax.ShapeDtypeStruct((n,), jnp.float32),
        ],
        mesh=mesh,
        scratch_types=[
            pltpu.VMEM((_BLK,), jnp.int32),
            pltpu.VMEM((_BLK, _HID), jnp.float32),
            pltpu.VMEM((_BLK,), jnp.float32),
            pltpu.VMEM((_VOCAB,), jnp.float32),
            pltpu.SemaphoreType.DMA,
        ],
        compiler_params=pltpu.CompilerParams(use_tc_tiling_on_sc=False,
                                             needs_layout_passes=False),
    )(seq1d, embed_table, sig1d)

    h = h_flat.reshape(B, L, _HID)
    soft = soft1d.reshape(B, L)
    return (soft, h)


# speedup vs baseline: 3.3323x; 1.0013x over previous
import jax
import jax.numpy as jnp
from jax import lax
from jax.experimental import pallas as pl
from jax.experimental.pallas import tpu as pltpu
from jax.experimental.pallas import tpu_sc as plsc

_VOCAB = 128
_HID = 64
_BLK = 800
_NC = 2
_NS = 16
_NW = _NC * _NS


def _gate_table_body(table_ref, w_ref, b_ref, sig_ref):
    t = table_ref[...]
    w = w_ref[...]
    logits = jnp.sum(t * w, axis=1) + b_ref[0, 0]
    sig_ref[...] = jax.nn.sigmoid(logits)[None, :]


def _sc_body(seq_hbm, table_hbm, sig_hbm, h_hbm, soft_hbm,
             idx_v, rows_v, soft_v, sig_v,
             sem_i0, sem_i1, sem_g0, sem_g1,
             sem_wh0, sem_wh1, sem_ws0, sem_ws1):
    wid = lax.axis_index("s") * _NC + lax.axis_index("c")
    n_idx = seq_hbm.shape[0]
    per_w = n_idx // _NW
    n_blk = per_w // _BLK
    base0 = wid * per_w

    sem_i = (sem_i0, sem_i1)
    sem_g = (sem_g0, sem_g1)
    sem_wh = (sem_wh0, sem_wh1)
    sem_ws = (sem_ws0, sem_ws1)

    pltpu.sync_copy(sig_hbm, sig_v)
    for q in (0, 1):
        pltpu.async_copy(seq_hbm.at[pl.ds(base0 + q * _BLK, _BLK)],
                         idx_v.at[q], sem_i[q])

    def pair_body(j, carry):
        for q in (0, 1):
            b = 2 * j + q
            pltpu.make_async_copy(seq_hbm.at[pl.ds(0, _BLK)],
                                  idx_v.at[q], sem_i[q]).wait()

            @pl.when(j > 0)
            def _():
                pltpu.make_async_copy(
                    rows_v.at[q], h_hbm.at[pl.ds(0, _BLK)], sem_wh[q]).wait()
                pltpu.make_async_copy(
                    soft_v.at[q], soft_hbm.at[pl.ds(0, _BLK)], sem_ws[q]).wait()

            g = pltpu.async_copy(table_hbm.at[idx_v.at[q]],
                                 rows_v.at[q], sem_g[q])

            for t in range(_BLK // 16):
                iv = idx_v[q, pl.ds(t * 16, 16)]
                soft_v[q, pl.ds(t * 16, 16)] = plsc.load_gather(sig_v, [iv])

            g.wait()

            nxt = jnp.minimum(base0 + (b + 2) * _BLK, base0 + per_w - _BLK)
            pltpu.async_copy(seq_hbm.at[pl.ds(nxt, _BLK)],
                             idx_v.at[q], sem_i[q])

            out0 = base0 + b * _BLK
            pltpu.async_copy(rows_v.at[q], h_hbm.at[pl.ds(out0, _BLK)],
                             sem_wh[q])
            pltpu.async_copy(soft_v.at[q], soft_hbm.at[pl.ds(out0, _BLK)],
                             sem_ws[q])
        return carry

    lax.fori_loop(0, n_blk // 2, pair_body, 0)

    for q in (0, 1):
        pltpu.make_async_copy(seq_hbm.at[pl.ds(0, _BLK)],
                              idx_v.at[q], sem_i[q]).wait()
        pltpu.make_async_copy(rows_v.at[q], h_hbm.at[pl.ds(0, _BLK)],
                              sem_wh[q]).wait()
        pltpu.make_async_copy(soft_v.at[q], soft_hbm.at[pl.ds(0, _BLK)],
                              sem_ws[q]).wait()


def kernel(seq, embed_table, gate_w, gate_b):
    B, L = seq.shape
    n = B * L
    seq1d = seq.reshape(n).astype(jnp.int32)

    sig = pl.pallas_call(
        _gate_table_body,
        out_shape=jax.ShapeDtypeStruct((1, _VOCAB), jnp.float32),
    )(embed_table, gate_w, gate_b.reshape(1, 1))
    sig1d = sig.reshape(_VOCAB)

    mesh = plsc.VectorSubcoreMesh(core_axis_name="c", subcore_axis_name="s",
                                  num_cores=_NC, num_subcores=_NS)
    h_flat, soft1d = pl.kernel(
        _sc_body,
        out_type=[
            jax.ShapeDtypeStruct((n, _HID), jnp.float32),
            jax.ShapeDtypeStruct((n,), jnp.float32),
        ],
        mesh=mesh,
        scratch_types=[
            pltpu.VMEM((2, _BLK), jnp.int32),
            pltpu.VMEM((2, _BLK, _HID), jnp.float32),
            pltpu.VMEM((2, _BLK), jnp.float32),
            pltpu.VMEM((_VOCAB,), jnp.float32),
        ] + [pltpu.SemaphoreType.DMA] * 8,
        compiler_params=pltpu.CompilerParams(use_tc_tiling_on_sc=False,
                                             needs_layout_passes=False),
    )(seq1d, embed_table, sig1d)

    h = h_flat.reshape(B, L, _HID)
    soft = soft1d.reshape(B, L)
    return (soft, h)
